# Initial kernel scaffold; baseline (speedup 1.0000x reference)
import jax
import jax.numpy as jnp
from jax.experimental import pallas as pl


def kernel(features, lengths, Wi, bi):
    raise NotImplementedError("write your pallas kernel here")


# speedup vs baseline: 2.3455x; 2.3455x over previous
import functools

import jax
import jax.numpy as jnp
from jax.experimental import pallas as pl
from jax.experimental.pallas import tpu as pltpu

B, K, D = 128, 576, 768
DP = 1024
KB = 64


def _theta_kernel(lengths_ref, logits_ref, theta_ref, thetam_ref):
    logits = logits_ref[...]
    m = jnp.max(logits)
    e = jnp.exp(logits - m)
    theta = e / jnp.sum(e)
    theta_ref[...] = theta
    lmax = jnp.minimum(jnp.max(lengths_ref[...]), K)
    ki = jax.lax.broadcasted_iota(jnp.int32, (K, 1), 0)
    thetam_ref[...] = jnp.where(ki < lmax, theta, 0.0)


def _sort_pool_kernel(feats_ref, thetam_ref, out_ref):
    kc = pl.program_id(1)
    x = feats_ref[0]
    x = jnp.concatenate([x, jnp.zeros((KB, DP - D), jnp.float32)], axis=1)

    lane = jax.lax.broadcasted_iota(jnp.int32, (KB, DP), 1)
    for level in range(1, 11):
        asc = (jax.lax.shift_right_logical(lane, level) & 1) == 1
        for d in (2 ** s for s in range(level - 1, -1, -1)):
            upper = (lane & d) != 0
            ctrl = jnp.logical_xor(jnp.logical_not(upper), asc)
            p = jnp.where(upper, pltpu.roll(x, d, 1), pltpu.roll(x, DP - d, 1))
            my = jnp.abs(x)
            pk = jnp.abs(p)
            lt1 = my < pk
            lt2 = pk < my
            swap = jnp.logical_or(jnp.logical_and(ctrl, lt1),
                                  jnp.logical_and(jnp.logical_not(ctrl), lt2))
            x = jnp.where(swap, p, x)

    vals = x[:, :D]
    w = thetam_ref[...]
    acc = jnp.sum(vals * w, axis=0, keepdims=True)[None]

    @pl.when(kc == 0)
    def _():
        out_ref[...] = acc

    @pl.when(kc != 0)
    def _():
        out_ref[...] += acc


def kernel(features, lengths, Wi, bi):
    a = jnp.arange(K, dtype=jnp.float32)[:, None]
    cat = jnp.concatenate((a, a[::-1]), axis=1)
    logits = cat @ Wi.T + bi

    theta, thetam = pl.pallas_call(
        _theta_kernel,
        out_shape=(
            jax.ShapeDtypeStruct((K, 1), jnp.float32),
            jax.ShapeDtypeStruct((K, 1), jnp.float32),
        ),
    )(lengths, logits)

    pooled = pl.pallas_call(
        _sort_pool_kernel,
        grid=(B, K // KB),
        in_specs=[
            pl.BlockSpec((1, KB, D), lambda b, kc: (b, kc, 0)),
            pl.BlockSpec((KB, 1), lambda b, kc: (kc, 0)),
        ],
        out_specs=pl.BlockSpec((1, 1, D), lambda b, kc: (b, 0, 0)),
        out_shape=jax.ShapeDtypeStruct((B, 1, D), jnp.float32),
    )(features, thetam)

    return (pooled.reshape(B, D), theta)


# speedup vs baseline: 5.8497x; 2.4940x over previous
import jax
import jax.numpy as jnp
from jax.experimental import pallas as pl
from jax.experimental.pallas import tpu as pltpu

B, K, D = 128, 576, 768
DP = 1024
R = 128
NBLK = (B * K) // R

_PHYS = [3, 4, 5, 6, 7, 8, 9, 0, 1, 2]


def _theta_kernel(lengths_ref, logits_ref, theta_ref, thetam_ref):
    logits = logits_ref[...]
    m = jnp.max(logits)
    e = jnp.exp(logits - m)
    theta = e / jnp.sum(e)
    theta_ref[...] = theta
    lmax = jnp.minimum(jnp.max(lengths_ref[...]), K)
    ki = jax.lax.broadcasted_iota(jnp.int32, (K, 1), 0)
    thetam_ref[...] = jnp.where(ki < lmax, theta, 0.0)


def _hilo(a, b):
    ge = jnp.abs(a) >= jnp.abs(b)
    return jnp.where(ge, a, b), jnp.where(ge, b, a)


def _stage_aligned(x, level, pt):
    d = 1 << pt
    if level == 10:
        y = x.reshape(DP // (2 * d), 2, d, 128)
        hi, lo = _hilo(y[:, 0], y[:, 1])
        return jnp.stack([hi, lo], axis=1).reshape(DP, 128)
    pl_ = _PHYS[level]
    if pl_ >= 3:
        g = 1 << (pl_ - pt - 1)
        y = x.reshape(-1, 2, g, 2, d, 128)
        a0, b0 = y[:, 0, :, 0], y[:, 0, :, 1]
        a1, b1 = y[:, 1, :, 0], y[:, 1, :, 1]
        hi0, lo0 = _hilo(a0, b0)
        hi1, lo1 = _hilo(a1, b1)
        dsc = jnp.stack([hi0, lo0], axis=2)
        asc = jnp.stack([lo1, hi1], axis=2)
        return jnp.stack([dsc, asc], axis=1).reshape(DP, 128)
    y = x.reshape(DP // (2 * d), 2, d, 128)
    a, b = y[:, 0], y[:, 1]
    hi, lo = _hilo(a, b)
    si = jax.lax.broadcasted_iota(jnp.int32, a.shape, 1) & 7
    dm = ((si >> pl_) & 1) == 0
    out_a = jnp.where(dm, hi, lo)
    out_b = jnp.where(dm, lo, hi)
    return jnp.stack([out_a, out_b], axis=1).reshape(DP, 128)


def _stage_sublane(x, level, pt):
    d = 1 << pt
    z = x.reshape(DP // 8, 8, 128)
    pieces = []
    for base in range(0, 8, 2 * d):
        a = z[:, base:base + d]
        b = z[:, base + d:base + 2 * d]
        hi, lo = _hilo(a, b)
        if level == 10:
            desc = True
        else:
            desc = ((base >> _PHYS[level]) & 1) == 0
        pieces += [hi, lo] if desc else [lo, hi]
    return jnp.concatenate(pieces, axis=1).reshape(DP, 128)


def _sort_pool_kernel(feats_ref, thetam_ref, out_ref):
    j = pl.program_id(0)
    x = feats_ref[...].T
    x = jnp.concatenate([x, jnp.zeros((DP - D, R), jnp.float32)], axis=0)

    for level in range(1, 11):
        for t in range(level - 1, -1, -1):
            pt = _PHYS[t]
            if pt >= 3:
                x = _stage_aligned(x, level, pt)
            else:
                x = _stage_sublane(x, level, pt)

    th = thetam_ref[...]
    lane_glob = j * R + jax.lax.broadcasted_iota(jnp.int32, (R, B), 0)
    lower = K * jax.lax.broadcasted_iota(jnp.int32, (R, B), 1)
    oh = jnp.logical_and(lane_glob >= lower, lane_glob < lower + K)
    w = oh.astype(jnp.float32) * th
    acc = jax.lax.dot(x, w, precision=jax.lax.Precision.HIGHEST,
                      preferred_element_type=jnp.float32)

    @pl.when(j == 0)
    def _():
        out_ref[...] = acc

    @pl.when(j != 0)
    def _():
        out_ref[...] += acc


def kernel(features, lengths, Wi, bi):
    a = jnp.arange(K, dtype=jnp.float32)[:, None]
    cat = jnp.concatenate((a, a[::-1]), axis=1)
    logits = cat @ Wi.T + bi

    theta, thetam = pl.pallas_call(
        _theta_kernel,
        out_shape=(
            jax.ShapeDtypeStruct((K, 1), jnp.float32),
            jax.ShapeDtypeStruct((K, 1), jnp.float32),
        ),
    )(lengths, logits)

    feats_flat = features.reshape(B * K, D)
    thetam_flat = jnp.tile(thetam, (B, 1))

    acc = pl.pallas_call(
        _sort_pool_kernel,
        grid=(NBLK,),
        in_specs=[
            pl.BlockSpec((R, D), lambda j: (j, 0)),
            pl.BlockSpec((R, 1), lambda j: (j, 0)),
        ],
        out_specs=pl.BlockSpec((DP, B), lambda j: (0, 0)),
        out_shape=jax.ShapeDtypeStruct((DP, B), jnp.float32),
    )(feats_flat, thetam_flat)

    pooled = acc.reshape(R, 8, B).transpose(1, 0, 2).reshape(DP, B)[:D].T
    return (pooled, theta)


# speedup vs baseline: 7.6282x; 1.3040x over previous
import jax
import jax.numpy as jnp
from jax.experimental import pallas as pl
from jax.experimental.pallas import tpu as pltpu

B, K, D = 128, 576, 768
DP = 1024
R = 128
NBLK = (B * K) // R
STRIP = 64

_PHYS = [3, 4, 5, 6, 7, 8, 9, 0, 1, 2]


def _theta_kernel(lengths_ref, logits_ref, theta_ref, thetam_ref):
    logits = logits_ref[...]
    m = jnp.max(logits)
    e = jnp.exp(logits - m)
    theta = e / jnp.sum(e)
    theta_ref[...] = theta
    lmax = jnp.minimum(jnp.max(lengths_ref[...]), K)
    ki = jax.lax.broadcasted_iota(jnp.int32, (K, 1), 0)
    thetam_ref[...] = jnp.where(ki < lmax, theta, 0.0)


def _ikey(v):
    return jax.lax.bitcast_convert_type(v, jnp.int32) & 0x7FFFFFFF


def _cmpex(a, b, flip):
    ka, kb = _ikey(a), _ikey(b)
    if flip is True:
        ka, kb = kb, ka
    elif flip is not None:
        ka, kb = ka ^ flip, kb ^ flip
    ge = ka >= kb
    return jnp.where(ge, a, b), jnp.where(ge, b, a)


def _stage_aligned(src, dst, level, pt):
    d = 1 << pt
    pl_ = None if level == 10 else _PHYS[level]
    for base in range(0, DP, 2 * d):
        if pl_ is None:
            flip = None
        elif pl_ >= 3:
            flip = True if ((base >> pl_) & 1) else None
        else:
            flip = "sub"
        for off in range(0, d, STRIP):
            w = min(STRIP, d)
            a = src[base + off:base + off + w]
            b = src[base + d + off:base + d + off + w]
            if flip == "sub":
                si = jax.lax.broadcasted_iota(jnp.int32, (w, 128), 0) & 7
                fm = jnp.where(((si >> pl_) & 1) == 1, -1, 0)
                out_a, out_b = _cmpex(a, b, fm)
            else:
                out_a, out_b = _cmpex(a, b, flip)
            dst[base + off:base + off + w] = out_a
            dst[base + d + off:base + d + off + w] = out_b


def _stage_sublane(src, dst, level, pt):
    d = 1 << pt
    z = src[...].reshape(DP // 8, 8, 128)
    pieces = []
    for base in range(0, 8, 2 * d):
        a = z[:, base:base + d]
        b = z[:, base + d:base + 2 * d]
        desc = True if level == 10 else ((base >> _PHYS[level]) & 1) == 0
        hi, lo = _cmpex(a, b, None)
        pieces += [hi, lo] if desc else [lo, hi]
    dst[...] = jnp.concatenate(pieces, axis=1).reshape(DP, 128)


def _sort_pool_kernel(feats_ref, thetam_ref, out_ref, s0, s1):
    j = pl.program_id(0)
    s0[0:D] = feats_ref[...].T
    s0[D:DP] = jnp.zeros((DP - D, R), jnp.float32)

    src, dst = s0, s1
    for level in range(1, 11):
        for t in range(level - 1, -1, -1):
            pt = _PHYS[t]
            if pt >= 3:
                _stage_aligned(src, dst, level, pt)
            else:
                _stage_sublane(src, dst, level, pt)
            src, dst = dst, src

    th = thetam_ref[...]
    lane_glob = j * R + jax.lax.broadcasted_iota(jnp.int32, (R, B), 0)
    lower = K * jax.lax.broadcasted_iota(jnp.int32, (R, B), 1)
    oh = jnp.logical_and(lane_glob >= lower, lane_glob < lower + K)
    w = oh.astype(jnp.float32) * th
    acc = jax.lax.dot(src[...], w, precision=jax.lax.Precision.HIGHEST,
                      preferred_element_type=jnp.float32)

    @pl.when(j == 0)
    def _():
        out_ref[...] = acc

    @pl.when(j != 0)
    def _():
        out_ref[...] += acc


def kernel(features, lengths, Wi, bi):
    a = jnp.arange(K, dtype=jnp.float32)[:, None]
    cat = jnp.concatenate((a, a[::-1]), axis=1)
    logits = cat @ Wi.T + bi

    theta, thetam = pl.pallas_call(
        _theta_kernel,
        out_shape=(
            jax.ShapeDtypeStruct((K, 1), jnp.float32),
            jax.ShapeDtypeStruct((K, 1), jnp.float32),
        ),
    )(lengths, logits)

    feats_flat = features.reshape(B * K, D)
    thetam_flat = jnp.tile(thetam, (B, 1))

    acc = pl.pallas_call(
        _sort_pool_kernel,
        grid=(NBLK,),
        in_specs=[
            pl.BlockSpec((R, D), lambda j: (j, 0)),
            pl.BlockSpec((R, 1), lambda j: (j, 0)),
        ],
        out_specs=pl.BlockSpec((DP, B), lambda j: (0, 0)),
        out_shape=jax.ShapeDtypeStruct((DP, B), jnp.float32),
        scratch_shapes=[
            pltpu.VMEM((DP, R), jnp.float32),
            pltpu.VMEM((DP, R), jnp.float32),
        ],
    )(feats_flat, thetam_flat)

    pooled = acc.reshape(R, 8, B).transpose(1, 0, 2).reshape(DP, B)[:D].T
    return (pooled, theta)


# speedup vs baseline: 10.5027x; 1.3768x over previous
import jax
import jax.numpy as jnp
from jax.experimental import pallas as pl
from jax.experimental.pallas import tpu as pltpu

B, K, D = 128, 576, 768
DP = 1024
R = 128
NBLK = (B * K) // R
STRIP = 64

_PHYS = [3, 4, 5, 6, 7, 8, 9, 0, 1, 2]


def _theta_kernel(lengths_ref, logits_ref, theta_ref, thetam_ref):
    logits = logits_ref[...]
    m = jnp.max(logits)
    e = jnp.exp(logits - m)
    theta = e / jnp.sum(e)
    theta_ref[...] = theta
    lmax = jnp.minimum(jnp.max(lengths_ref[...]), K)
    ki = jax.lax.broadcasted_iota(jnp.int32, (K, 1), 0)
    thetam_ref[...] = jnp.where(ki < lmax, theta, 0.0)


def _ikey(v):
    return jax.lax.bitcast_convert_type(v, jnp.int32) & 0x7FFFFFFF


def _cmpex(a, b, flip):
    ka, kb = _ikey(a), _ikey(b)
    if flip is True:
        ka, kb = kb, ka
    elif flip is not None:
        ka, kb = ka ^ flip, kb ^ flip
    ge = ka >= kb
    return jnp.where(ge, a, b), jnp.where(ge, b, a)


def _stage_aligned(src, dst, level, pt):
    d = 1 << pt
    pl_ = None if level == 10 else _PHYS[level]
    for base in range(0, DP, 2 * d):
        if pl_ is None:
            flip = None
        elif pl_ >= 3:
            flip = True if ((base >> pl_) & 1) else None
        else:
            flip = "sub"
        for off in range(0, d, STRIP):
            w = min(STRIP, d)
            a = src[base + off:base + off + w]
            b = src[base + d + off:base + d + off + w]
            if flip == "sub":
                si = jax.lax.broadcasted_iota(jnp.int32, (w, 128), 0) & 7
                fm = jnp.where(((si >> pl_) & 1) == 1, -1, 0)
                out_a, out_b = _cmpex(a, b, fm)
            else:
                out_a, out_b = _cmpex(a, b, flip)
            dst[base + off:base + off + w] = out_a
            dst[base + d + off:base + d + off + w] = out_b


def _stage_sublane_group(src, dst, level, ts):
    z = src[...].reshape(DP // 8, 8, 128)
    si = jax.lax.broadcasted_iota(jnp.int32, (DP // 8, 8, 128), 1)
    pl_ = None if level == 10 else _PHYS[level]
    asc = None if pl_ is None else (si & (1 << pl_)) != 0
    for t in ts:
        d = 1 << _PHYS[t]
        up = (si & d) != 0
        r1 = pltpu.roll(z, d, 1)
        r2 = pltpu.roll(z, 8 - d, 1)
        p = jnp.where(up, r1, r2)
        wants_min = up if asc is None else jnp.logical_xor(up, asc)
        fm = jnp.where(wants_min, -1, 0)
        take = (_ikey(p) ^ fm) > (_ikey(z) ^ fm)
        z = jnp.where(take, p, z)
    dst[...] = z.reshape(DP, 128)


def _sort_pool_kernel(feats_ref, thetam_ref, out_ref, s0, s1):
    j = pl.program_id(0)
    s0[0:D] = feats_ref[...].T
    s0[D:DP] = jnp.zeros((DP - D, R), jnp.float32)

    src, dst = s0, s1
    for level in range(1, 11):
        sub_ts = [t for t in range(level - 1, -1, -1) if _PHYS[t] < 3]
        if sub_ts:
            _stage_sublane_group(src, dst, level, sub_ts)
            src, dst = dst, src
        for t in range(level - 1, -1, -1):
            pt = _PHYS[t]
            if pt >= 3:
                _stage_aligned(src, dst, level, pt)
                src, dst = dst, src

    th = thetam_ref[...]
    lane_glob = j * R + jax.lax.broadcasted_iota(jnp.int32, (R, B), 0)
    lower = K * jax.lax.broadcasted_iota(jnp.int32, (R, B), 1)
    oh = jnp.logical_and(lane_glob >= lower, lane_glob < lower + K)
    w = oh.astype(jnp.float32) * th
    acc = jax.lax.dot(src[...], w, precision=jax.lax.Precision.HIGHEST,
                      preferred_element_type=jnp.float32)

    @pl.when(j == 0)
    def _():
        out_ref[...] = acc

    @pl.when(j != 0)
    def _():
        out_ref[...] += acc


def kernel(features, lengths, Wi, bi):
    a = jnp.arange(K, dtype=jnp.float32)[:, None]
    cat = jnp.concatenate((a, a[::-1]), axis=1)
    logits = cat @ Wi.T + bi

    theta, thetam = pl.pallas_call(
        _theta_kernel,
        out_shape=(
            jax.ShapeDtypeStruct((K, 1), jnp.float32),
            jax.ShapeDtypeStruct((K, 1), jnp.float32),
        ),
    )(lengths, logits)

    feats_flat = features.reshape(B * K, D)
    thetam_flat = jnp.tile(thetam, (B, 1))

    acc = pl.pallas_call(
        _sort_pool_kernel,
        grid=(NBLK,),
        in_specs=[
            pl.BlockSpec((R, D), lambda j: (j, 0)),
            pl.BlockSpec((R, 1), lambda j: (j, 0)),
        ],
        out_specs=pl.BlockSpec((DP, B), lambda j: (0, 0)),
        out_shape=jax.ShapeDtypeStruct((DP, B), jnp.float32),
        scratch_shapes=[
            pltpu.VMEM((DP, R), jnp.float32),
            pltpu.VMEM((DP, R), jnp.float32),
        ],
    )(feats_flat, thetam_flat)

    pooled = acc.reshape(R, 8, B).transpose(1, 0, 2).reshape(DP, B)[:D].T
    return (pooled, theta)


# speedup vs baseline: 10.5329x; 1.0029x over previous
import jax
import jax.numpy as jnp
from jax.experimental import pallas as pl
from jax.experimental.pallas import tpu as pltpu

B, K, D = 128, 576, 768
DP = 1024
R = 128
NBLK = (B * K) // R
STRIP = 64

_PHYS = [3, 4, 5, 6, 7, 8, 9, 0, 1, 2]


def _theta_kernel(lengths_ref, logits_ref, theta_ref, thetam_ref):
    logits = logits_ref[...]
    m = jnp.max(logits)
    e = jnp.exp(logits - m)
    theta = e / jnp.sum(e)
    theta_ref[...] = theta
    lmax = jnp.minimum(jnp.max(lengths_ref[...]), K)
    ki = jax.lax.broadcasted_iota(jnp.int32, (K, 1), 0)
    thetam_ref[...] = jnp.where(ki < lmax, theta, 0.0)


def _ikey(v):
    return jax.lax.bitcast_convert_type(v, jnp.int32) & 0x7FFFFFFF


def _cmpex(a, b, flip):
    ka, kb = _ikey(a), _ikey(b)
    if flip is True:
        ka, kb = kb, ka
    elif flip is not None:
        ka, kb = ka ^ flip, kb ^ flip
    ge = ka >= kb
    return jnp.where(ge, a, b), jnp.where(ge, b, a)


def _stage_aligned(src, dst, level, pt):
    d = 1 << pt
    pl_ = None if level == 10 else _PHYS[level]
    for base in range(0, DP, 2 * d):
        if level <= 5 and base >= D:
            dst[base:base + 2 * d] = jnp.zeros((2 * d, 128), jnp.float32)
            continue
        if pl_ is None:
            flip = None
        elif pl_ >= 3:
            flip = True if ((base >> pl_) & 1) else None
        else:
            flip = "sub"
        for off in range(0, d, STRIP):
            w = min(STRIP, d)
            a = src[base + off:base + off + w]
            b = src[base + d + off:base + d + off + w]
            if flip == "sub":
                si = jax.lax.broadcasted_iota(jnp.int32, (w, 128), 0) & 7
                inv = ((si >> pl_) & 1) == 1
                ge = jnp.logical_xor(_ikey(a) >= _ikey(b), inv)
                out_a = jnp.where(ge, a, b)
                out_b = jnp.where(ge, b, a)
            else:
                out_a, out_b = _cmpex(a, b, flip)
            dst[base + off:base + off + w] = out_a
            dst[base + d + off:base + d + off + w] = out_b


def _stage_sublane_group(src, dst, level, ts):
    z = src[...].reshape(DP // 8, 8, 128)
    si = jax.lax.broadcasted_iota(jnp.int32, (DP // 8, 8, 128), 1)
    pl_ = None if level == 10 else _PHYS[level]
    asc = None if pl_ is None else (si & (1 << pl_)) != 0
    for t in ts:
        d = 1 << _PHYS[t]
        up = (si & d) != 0
        r1 = pltpu.roll(z, d, 1)
        r2 = pltpu.roll(z, 8 - d, 1)
        p = jnp.where(up, r1, r2)
        wants_min = up if asc is None else jnp.logical_xor(up, asc)
        fm = jnp.where(wants_min, -1, 0)
        take = (_ikey(p) ^ fm) > (_ikey(z) ^ fm)
        z = jnp.where(take, p, z)
    dst[...] = z.reshape(DP, 128)


def _sort_pool_kernel(feats_ref, thetam_ref, out_ref, s0, s1):
    j = pl.program_id(0)
    s0[0:D] = feats_ref[...].T
    s0[D:DP] = jnp.zeros((DP - D, R), jnp.float32)

    src, dst = s0, s1
    for level in range(1, 11):
        sub_ts = [t for t in range(level - 1, -1, -1) if _PHYS[t] < 3]
        if sub_ts:
            _stage_sublane_group(src, dst, level, sub_ts)
            src, dst = dst, src
        for t in range(level - 1, -1, -1):
            pt = _PHYS[t]
            if pt >= 3:
                _stage_aligned(src, dst, level, pt)
                src, dst = dst, src

    th = thetam_ref[...]
    lane_glob = j * R + jax.lax.broadcasted_iota(jnp.int32, (R, B), 0)
    lower = K * jax.lax.broadcasted_iota(jnp.int32, (R, B), 1)
    oh = jnp.logical_and(lane_glob >= lower, lane_glob < lower + K)
    w = oh.astype(jnp.float32) * th
    acc = jax.lax.dot(src[...], w, precision=jax.lax.Precision.HIGHEST,
                      preferred_element_type=jnp.float32)

    @pl.when(j == 0)
    def _():
        out_ref[...] = acc

    @pl.when(j != 0)
    def _():
        out_ref[...] += acc


def kernel(features, lengths, Wi, bi):
    a = jnp.arange(K, dtype=jnp.float32)[:, None]
    cat = jnp.concatenate((a, a[::-1]), axis=1)
    logits = cat @ Wi.T + bi

    theta, thetam = pl.pallas_call(
        _theta_kernel,
        out_shape=(
            jax.ShapeDtypeStruct((K, 1), jnp.float32),
            jax.ShapeDtypeStruct((K, 1), jnp.float32),
        ),
    )(lengths, logits)

    feats_flat = features.reshape(B * K, D)
    thetam_flat = jnp.tile(thetam, (B, 1))

    acc = pl.pallas_call(
        _sort_pool_kernel,
        grid=(NBLK,),
        in_specs=[
            pl.BlockSpec((R, D), lambda j: (j, 0)),
            pl.BlockSpec((R, 1), lambda j: (j, 0)),
        ],
        out_specs=pl.BlockSpec((DP, B), lambda j: (0, 0)),
        out_shape=jax.ShapeDtypeStruct((DP, B), jnp.float32),
        scratch_shapes=[
            pltpu.VMEM((DP, R), jnp.float32),
            pltpu.VMEM((DP, R), jnp.float32),
        ],
    )(feats_flat, thetam_flat)

    pooled = acc.reshape(R, 8, B).transpose(1, 0, 2).reshape(DP, B)[:D].T
    return (pooled, theta)
